# baseline (speedup 1.0000x reference)
import functools

import jax
import jax.numpy as jnp
from jax import lax
from jax.experimental import pallas as pl
from jax.experimental.pallas import tpu as pltpu
from jax.experimental.pallas import tpu_sc as plsc

_V = 1_000_000
_D = 64
_NB = 16384
_S = 50
_NC = 2
_NS = 16
_D_PER_C = _D // _NC
_B_PER_T = _NB // _NS
_SG = 5
_NG = _S // _SG
_IDXW = _S * _B_PER_T
_VP = 1_000_064
_CH = 66560
_C15 = _V - 64 - 15 * _CH
_GW = _SG * _B_PER_T

_mesh = plsc.VectorSubcoreMesh(core_axis_name="c", subcore_axis_name="s")


@functools.partial(
    pl.kernel,
    mesh=_mesh,
    out_type=jax.ShapeDtypeStruct((_S, _D, _NB), jnp.float32),
    scratch_types=[
        pltpu.VMEM((1, _IDXW), jnp.int32),
        pltpu.VMEM((1, _GW), jnp.float32),
        pltpu.VMEM((1, _GW), jnp.float32),
        pltpu.VMEM_SHARED((1, _VP), jnp.float32),
        pltpu.SemaphoreType.DMA,
        pltpu.SemaphoreType.DMA,
        pltpu.SemaphoreType.DMA,
        pltpu.SemaphoreType.DMA,
        pltpu.SemaphoreType.DMA,
    ],
)
def _emb_lookup(idx_hbm, wt_hbm, wtail_hbm, out_hbm, idx_v, gbuf0, gbuf1,
                row, sem_sa, sem_g, sem_g1, sem_t0, sem_t1):
    cid = lax.axis_index("c")
    tid = lax.axis_index("s")
    d_base = cid * _D_PER_C
    b0 = tid * _B_PER_T

    gbufs = (gbuf0, gbuf1)
    sem_ts = (sem_t0, sem_t1)

    for s in range(_S):
        pltpu.async_copy(idx_hbm.at[:, pl.ds(s * _NB + b0, _B_PER_T)],
                         idx_v.at[:, pl.ds(s * _B_PER_T, _B_PER_T)],
                         sem_g)
    pltpu.make_async_copy(idx_hbm.at[:, pl.ds(0, _IDXW)], idx_v,
                          sem_g).wait()

    def body(j, carry):
        dj = d_base + j
        c0 = tid * _CH

        @pl.when(tid < 15)
        def _():
            pltpu.async_copy(wt_hbm.at[pl.ds(dj, 1), pl.ds(c0, _CH)],
                             row.at[:, pl.ds(c0, _CH)], sem_sa)
            pltpu.make_async_copy(wt_hbm.at[pl.ds(0, 1), pl.ds(0, _CH)],
                                  row.at[:, pl.ds(0, _CH)], sem_sa).wait()

        @pl.when(tid == 15)
        def _():
            pltpu.async_copy(
                wt_hbm.at[pl.ds(dj, 1), pl.ds(15 * _CH, _C15)],
                row.at[:, pl.ds(15 * _CH, _C15)], sem_sa)
            pltpu.async_copy(wtail_hbm.at[pl.ds(dj, 1), :],
                             row.at[:, pl.ds(_V - 64, 128)], sem_sa)
            pltpu.make_async_copy(
                wt_hbm.at[pl.ds(0, 1), pl.ds(0, _C15)],
                row.at[:, pl.ds(0, _C15)], sem_sa).wait()
            pltpu.make_async_copy(
                wtail_hbm.at[pl.ds(0, 1), :],
                row.at[:, pl.ds(0, 128)], sem_sa).wait()
        plsc.subcore_barrier()
        dd = d_base + j
        sem_gs = (sem_g, sem_g1)

        def drain_store(h):
            pltpu.make_async_copy(
                wt_hbm.at[pl.ds(0, 1), pl.ds(0, _GW)],
                gbufs[h], sem_ts[h]).wait()

        def fire_gather(g, h):
            pltpu.async_copy(
                row.at[idx_v.at[:, pl.ds(g * _GW, _GW)]],
                gbufs[h], sem_gs[h])

        def wait_gather(h):
            pltpu.make_async_copy(
                wt_hbm.at[pl.ds(0, 1), pl.ds(0, _GW)],
                gbufs[h], sem_gs[h]).wait()

        def fire_store(g, h):
            pltpu.async_copy(
                gbufs[h].reshape(_SG, _B_PER_T),
                out_hbm.at[pl.ds(g * _SG, _SG), dd,
                           pl.ds(b0, _B_PER_T)],
                sem_ts[h])

        @pl.when(j >= 1)
        def _():
            drain_store(0)
        fire_gather(0, 0)
        for g in range(_NG):
            h = g % 2
            if g + 1 < _NG:
                nh = 1 - h
                if g + 1 >= 2:
                    drain_store(nh)
                else:
                    @pl.when(j >= 1)
                    def _():
                        drain_store(nh)
                fire_gather(g + 1, nh)
            wait_gather(h)
            fire_store(g, h)

    def body_with_tail_barrier(j, carry):
        body(j, carry)
        plsc.subcore_barrier()
        return carry

    lax.fori_loop(0, _D_PER_C, body_with_tail_barrier, 0)

    for h in range(2):
        pltpu.make_async_copy(
            wt_hbm.at[pl.ds(0, 1), pl.ds(0, _GW)],
            gbufs[h], sem_ts[h]).wait()


def kernel(token_ids, weight):
    idx_flat = token_ids.T.reshape(1, _S * _NB)
    wtail = jnp.pad(weight[_V - 64:].T, ((0, 0), (0, 64)))
    out_t = _emb_lookup(idx_flat, weight.T, wtail)
    return jnp.transpose(out_t, (2, 0, 1))


# speedup vs baseline: 1.0030x; 1.0030x over previous
import functools

import jax
import jax.numpy as jnp
from jax import lax
from jax.experimental import pallas as pl
from jax.experimental.pallas import tpu as pltpu
from jax.experimental.pallas import tpu_sc as plsc

_V = 1_000_000
_D = 64
_NB = 16384
_S = 50
_NC = 2
_NS = 16
_D_PER_C = _D // _NC
_B_PER_T = _NB // _NS
_SG = 5
_NG = _S // _SG
_IDXW = _S * _B_PER_T
_GW = _SG * _B_PER_T

_mesh = plsc.VectorSubcoreMesh(core_axis_name="c", subcore_axis_name="s")


@functools.partial(
    pl.kernel,
    mesh=_mesh,
    out_type=jax.ShapeDtypeStruct((_S, _D, _NB), jnp.float32),
    scratch_types=[
        pltpu.VMEM((1, _IDXW), jnp.int32),
        pltpu.VMEM((1, _GW), jnp.float32),
        pltpu.VMEM((1, _GW), jnp.float32),
        pltpu.VMEM_SHARED((1, _V), jnp.float32),
        pltpu.SemaphoreType.DMA,
        pltpu.SemaphoreType.DMA,
        pltpu.SemaphoreType.DMA,
        pltpu.SemaphoreType.DMA,
        pltpu.SemaphoreType.DMA,
    ],
)
def _emb_lookup(idx_hbm, wt_hbm, out_hbm, idx_v, gbuf0, gbuf1,
                row, sem_sa, sem_g, sem_g1, sem_t0, sem_t1):
    cid = lax.axis_index("c")
    tid = lax.axis_index("s")
    d_base = cid * _D_PER_C
    b0 = tid * _B_PER_T

    gbufs = (gbuf0, gbuf1)
    sem_ts = (sem_t0, sem_t1)

    for s in range(_S):
        pltpu.async_copy(idx_hbm.at[:, pl.ds(s * _NB + b0, _B_PER_T)],
                         idx_v.at[:, pl.ds(s * _B_PER_T, _B_PER_T)],
                         sem_g)
    pltpu.make_async_copy(idx_hbm.at[:, pl.ds(0, _IDXW)], idx_v,
                          sem_g).wait()

    def stage(j, row, sem):
        pltpu.async_copy(wt_hbm.at[pl.ds(d_base + j, 1), :], row, sem)

    def wait_stage(row, sem):
        pltpu.make_async_copy(wt_hbm.at[pl.ds(0, 1), :], row, sem).wait()

    def body(j, carry):
        @pl.when(tid == 0)
        def _():
            stage(j, row, sem_sa)
            wait_stage(row, sem_sa)
        plsc.subcore_barrier()
        dd = d_base + j
        sem_gs = (sem_g, sem_g1)

        def drain_store(h):
            pltpu.make_async_copy(
                wt_hbm.at[pl.ds(0, 1), pl.ds(0, _GW)],
                gbufs[h], sem_ts[h]).wait()

        def fire_gather(g, h):
            pltpu.async_copy(
                row.at[idx_v.at[:, pl.ds(g * _GW, _GW)]],
                gbufs[h], sem_gs[h])

        def wait_gather(h):
            pltpu.make_async_copy(
                wt_hbm.at[pl.ds(0, 1), pl.ds(0, _GW)],
                gbufs[h], sem_gs[h]).wait()

        def fire_store(g, h):
            pltpu.async_copy(
                gbufs[h].reshape(_SG, _B_PER_T),
                out_hbm.at[pl.ds(g * _SG, _SG), dd,
                           pl.ds(b0, _B_PER_T)],
                sem_ts[h])

        @pl.when(j >= 1)
        def _():
            drain_store(0)
        fire_gather(0, 0)
        for g in range(_NG):
            h = g % 2
            if g + 1 < _NG:
                nh = 1 - h
                if g + 1 >= 2:
                    drain_store(nh)
                else:
                    @pl.when(j >= 1)
                    def _():
                        drain_store(nh)
                fire_gather(g + 1, nh)
            wait_gather(h)
            fire_store(g, h)

    def body_with_tail_barrier(j, carry):
        body(j, carry)
        plsc.subcore_barrier()
        return carry

    lax.fori_loop(0, _D_PER_C, body_with_tail_barrier, 0)

    for h in range(2):
        pltpu.make_async_copy(
            wt_hbm.at[pl.ds(0, 1), pl.ds(0, _GW)],
            gbufs[h], sem_ts[h]).wait()


def kernel(token_ids, weight):
    idx_flat = token_ids.T.reshape(1, _S * _NB)
    out_t = _emb_lookup(idx_flat, weight.T)
    return jnp.transpose(out_t, (2, 0, 1))
